# speedup vs baseline: 2.2221x; 2.2221x over previous
import functools

import jax
import jax.numpy as jnp
from jax import lax
from jax.experimental import pallas as pl
from jax.experimental.pallas import tpu as pltpu
from jax.experimental.pallas import tpu_sc as plsc

NUM_CORES = 2
NUM_SUBCORES = 16
NUM_WORKERS = NUM_CORES * NUM_SUBCORES

CHUNK = 32


def _make_lookup(n_rows: int, d: int):
    rows_per_w = n_rows // NUM_WORKERS
    nchunk = rows_per_w // CHUNK
    mesh = plsc.VectorSubcoreMesh(
        core_axis_name="c", subcore_axis_name="s",
        num_cores=NUM_CORES, num_subcores=NUM_SUBCORES,
    )

    @functools.partial(
        pl.kernel,
        out_type=jax.ShapeDtypeStruct((n_rows, d), jnp.float32),
        mesh=mesh,
        scratch_types=[
            pltpu.VMEM((nchunk, CHUNK), jnp.int32),
            pltpu.VMEM((CHUNK, d), jnp.float32),
            pltpu.VMEM((CHUNK, d), jnp.float32),
            pltpu.SemaphoreType.DMA,
            pltpu.SemaphoreType.DMA,
        ],
    )
    def run(idx_hbm, table_hbm, out_hbm, idx_v, buf0, buf1, sem0, sem1):
        wid = lax.axis_index("s") * NUM_CORES + lax.axis_index("c")
        base = wid * rows_per_w
        pltpu.sync_copy(idx_hbm.at[wid], idx_v)

        def pair(p, _):
            g0 = p * 2
            cp0 = pltpu.async_copy(table_hbm.at[idx_v.at[g0]], buf0, sem0)
            cp1 = pltpu.async_copy(table_hbm.at[idx_v.at[g0 + 1]], buf1, sem1)
            cp0.wait()
            pltpu.sync_copy(buf0, out_hbm.at[pl.ds(base + g0 * CHUNK, CHUNK)])
            cp1.wait()
            pltpu.sync_copy(
                buf1, out_hbm.at[pl.ds(base + (g0 + 1) * CHUNK, CHUNK)])
            return 0

        lax.fori_loop(0, nchunk // 2, pair, 0)

    return run


def kernel(positions, pos_emb_weight):
    b, s = positions.shape
    n_rows = b * s
    d = pos_emb_weight.shape[1]
    rows_per_w = n_rows // NUM_WORKERS
    idx = positions.reshape(NUM_WORKERS, rows_per_w // CHUNK, CHUNK)
    out = _make_lookup(n_rows, d)(idx.astype(jnp.int32), pos_emb_weight)
    return out.reshape(b, s, d)


# speedup vs baseline: 2.3762x; 1.0693x over previous
import functools

import jax
import jax.numpy as jnp
from jax import lax
from jax.experimental import pallas as pl
from jax.experimental.pallas import tpu as pltpu
from jax.experimental.pallas import tpu_sc as plsc

NUM_CORES = 2
NUM_SUBCORES = 16
NUM_WORKERS = NUM_CORES * NUM_SUBCORES

CHUNK = 16
NSLOT = 4


def _make_lookup(n_rows: int, d: int):
    rows_per_w = n_rows // NUM_WORKERS
    nchunk = rows_per_w // CHUNK
    mesh = plsc.VectorSubcoreMesh(
        core_axis_name="c", subcore_axis_name="s",
        num_cores=NUM_CORES, num_subcores=NUM_SUBCORES,
    )

    @functools.partial(
        pl.kernel,
        out_type=jax.ShapeDtypeStruct((n_rows, d), jnp.float32),
        mesh=mesh,
        scratch_types=(
            [pltpu.VMEM((nchunk, CHUNK), jnp.int32)]
            + [pltpu.VMEM((CHUNK, d), jnp.float32) for _ in range(NSLOT)]
            + [pltpu.SemaphoreType.DMA for _ in range(2 * NSLOT)]
        ),
    )
    def run(idx_hbm, table_hbm, out_hbm, idx_v, *rest):
        bufs = rest[:NSLOT]
        gsems = rest[NSLOT:2 * NSLOT]
        ssems = rest[2 * NSLOT:]
        wid = lax.axis_index("s") * NUM_CORES + lax.axis_index("c")
        base = wid * rows_per_w
        pltpu.sync_copy(idx_hbm.at[wid], idx_v)

        def gather(g, j):
            return pltpu.make_async_copy(
                table_hbm.at[idx_v.at[g]], bufs[j], gsems[j])

        def store(g, j):
            return pltpu.make_async_copy(
                bufs[j], out_hbm.at[pl.ds(base + g * CHUNK, CHUNK)], ssems[j])

        for j in range(NSLOT):
            gather(j, j).start()

        def body(p, _):
            g0 = p * NSLOT
            for j in range(NSLOT):
                g = g0 + j
                gather(g, j).wait()
                store(g, j).start()

                @pl.when(g + NSLOT < nchunk)
                def _():
                    store(g, j).wait()
                    gather(g + NSLOT, j).start()
            return 0

        lax.fori_loop(0, nchunk // NSLOT, body, 0)

        for j in range(NSLOT):
            store(nchunk - NSLOT + j, j).wait()

    return run


def kernel(positions, pos_emb_weight):
    b, s = positions.shape
    n_rows = b * s
    d = pos_emb_weight.shape[1]
    rows_per_w = n_rows // NUM_WORKERS
    idx = positions.reshape(NUM_WORKERS, rows_per_w // CHUNK, CHUNK)
    out = _make_lookup(n_rows, d)(idx.astype(jnp.int32), pos_emb_weight)
    return out.reshape(b, s, d)
